# baseline (speedup 1.0000x reference)
import functools
import math

import jax
import jax.numpy as jnp
import numpy as np
from jax import lax
from jax.experimental import pallas as pl
from jax.experimental.pallas import tpu as pltpu
from jax.experimental.pallas import tpu_sc as plsc

VOCAB = 100000
EMB_DIM = 1024
BATCH = 4
SEQ = 4096
SCALE = math.sqrt(float(EMB_DIM))

_NC = 2
_NS = 16
_NW = _NC * _NS
_TOK = BATCH * SEQ
_TPW = _TOK // _NW
_C = 32
_NCH = _TPW // _C
_LANES = EMB_DIM // 16


def _sinusoidal_pe_np(max_len, d_model):
    pos = np.arange(max_len, dtype=np.float32)[:, None]
    div = np.exp(
        np.arange(0, d_model, 2, dtype=np.float32) * (-math.log(10000.0) / d_model)
    )
    pe = np.zeros((max_len, d_model), dtype=np.float32)
    pe[:, 0::2] = np.sin(pos * div)
    pe[:, 1::2] = np.cos(pos * div)
    return pe


_PE = _sinusoidal_pe_np(SEQ, EMB_DIM)


def _body(x_hbm, table_hbm, pe_hbm, out_hbm, idx_v, rows_v, pe_v, sem):
    wid = lax.axis_index("s") * _NC + lax.axis_index("c")
    base = wid * _TPW

    def chunk(c, carry):
        tok_base = base + c * _C
        pos_base = lax.rem(tok_base, SEQ)
        pltpu.sync_copy(x_hbm.at[pl.ds(tok_base, _C)], idx_v)
        pltpu.async_copy(table_hbm.at[idx_v], rows_v, sem).wait()
        pltpu.sync_copy(pe_hbm.at[pl.ds(pos_base, _C)], pe_v)

        def fuse(k, carry2):
            i = k // _LANES
            j = (k % _LANES) * 16
            rows_v[i, pl.ds(j, 16)] = (
                rows_v[i, pl.ds(j, 16)] * SCALE + pe_v[i, pl.ds(j, 16)]
            )
            return carry2

        lax.fori_loop(0, _C * _LANES, fuse, 0)
        pltpu.sync_copy(rows_v, out_hbm.at[pl.ds(tok_base, _C)])
        return carry

    lax.fori_loop(0, _NCH, chunk, 0)


@functools.partial(jax.jit, static_argnames=())
def kernel(x, token_table):
    x_flat = x.reshape(-1).astype(jnp.int32)
    pe = jnp.asarray(_PE)
    mesh = plsc.VectorSubcoreMesh(core_axis_name="c", subcore_axis_name="s")
    run = functools.partial(
        pl.kernel,
        mesh=mesh,
        out_type=jax.ShapeDtypeStruct((_TOK, EMB_DIM), jnp.float32),
        scratch_types=[
            pltpu.VMEM((_C,), jnp.int32),
            pltpu.VMEM((_C, EMB_DIM), jnp.float32),
            pltpu.VMEM((_C, EMB_DIM), jnp.float32),
            pltpu.SemaphoreType.DMA,
        ],
    )(_body)
    out = run(x_flat, token_table, pe)
    return out.reshape(BATCH, SEQ, EMB_DIM)


# speedup vs baseline: 3.0950x; 3.0950x over previous
import functools
import math

import jax
import jax.numpy as jnp
import numpy as np
from jax import lax
from jax.experimental import pallas as pl
from jax.experimental.pallas import tpu as pltpu
from jax.experimental.pallas import tpu_sc as plsc

VOCAB = 100000
EMB_DIM = 1024
BATCH = 4
SEQ = 4096
SCALE = math.sqrt(float(EMB_DIM))

_NC = 2
_NS = 16
_NW = _NC * _NS
_PPW = SEQ // _NW
_C = 16
_NQ = _PPW // _C
_T = _NQ * BATCH
_LANES = EMB_DIM // 16


def _sinusoidal_pe_np(max_len, d_model):
    pos = np.arange(max_len, dtype=np.float32)[:, None]
    div = np.exp(
        np.arange(0, d_model, 2, dtype=np.float32) * (-math.log(10000.0) / d_model)
    )
    pe = np.zeros((max_len, d_model), dtype=np.float32)
    pe[:, 0::2] = np.sin(pos * div)
    pe[:, 1::2] = np.cos(pos * div)
    return pe


_PE = _sinusoidal_pe_np(SEQ, EMB_DIM)


def _body(x_hbm, table_hbm, pe_hbm, out_hbm,
          idx_v, pe0, pe1, rows0, rows1, outv0, outv1,
          gs0, gs1, ps0, ps1, ss0, ss1):
    wid = lax.axis_index("s") * _NC + lax.axis_index("c")
    pbase = wid * _PPW
    pe_bufs = (pe0, pe1)
    rows_bufs = (rows0, rows1)
    out_bufs = (outv0, outv1)
    gsem = (gs0, gs1)
    psem = (ps0, ps1)
    ssem = (ss0, ss1)

    for b in range(BATCH):
        pltpu.sync_copy(x_hbm.at[b, pl.ds(pbase, _PPW)], idx_v.at[b])

    def g_args(q, b, slot):
        return (table_hbm.at[idx_v.at[b, pl.ds(q * _C, _C)]],
                rows_bufs[slot], gsem[slot])

    def p_args(q, slot):
        return (pe_hbm.at[pl.ds(pbase + q * _C, _C)], pe_bufs[slot], psem[slot])

    def s_args(q, b, slot):
        tok = b * SEQ + pbase + q * _C
        return (out_bufs[slot], out_hbm.at[pl.ds(tok, _C)], ssem[slot])

    def fuse(rslot, pslot, oslot):
        rows, pe_b, out_b = rows_bufs[rslot], pe_bufs[pslot], out_bufs[oslot]

        def row(i, carry):
            for v in range(_LANES):
                j = v * 16
                out_b[i, pl.ds(j, 16)] = (
                    rows[i, pl.ds(j, 16)] * SCALE + pe_b[i, pl.ds(j, 16)]
                )
            return carry

        lax.fori_loop(0, _C, row, 0)

    pltpu.async_copy(*g_args(0, 0, 0))
    pltpu.async_copy(*g_args(0, 1, 1))
    pltpu.async_copy(*p_args(0, 0))

    def outer(i, carry):
        for qq in (0, 1):
            q = 2 * i + qq
            for b in range(BATCH):
                slot = b % 2
                if b == 0:
                    def issue_pe():
                        pltpu.async_copy(*p_args(q + 1, 1 - qq))
                    if qq == 0:
                        issue_pe()
                    else:
                        pl.when(i <= _NQ // 2 - 2)(issue_pe)
                    pltpu.make_async_copy(*p_args(q, qq)).wait()
                pltpu.make_async_copy(*g_args(q, b, slot)).wait()
                def wait_store():
                    pltpu.make_async_copy(*s_args(q, b, slot)).wait()
                if qq == 0 and b < 2:
                    pl.when(i >= 1)(wait_store)
                else:
                    wait_store()
                fuse(slot, qq, slot)
                pltpu.async_copy(*s_args(q, b, slot))
                def issue_gather():
                    if b < 2:
                        pltpu.async_copy(*g_args(q, b + 2, slot))
                    else:
                        pltpu.async_copy(*g_args(q + 1, b - 2, slot))
                if qq == 1 and b >= 2:
                    pl.when(i <= _NQ // 2 - 2)(issue_gather)
                else:
                    issue_gather()
        return carry

    lax.fori_loop(0, _NQ // 2, outer, 0)

    pltpu.make_async_copy(*s_args(_NQ - 1, 2, 0)).wait()
    pltpu.make_async_copy(*s_args(_NQ - 1, 3, 1)).wait()


@jax.jit
def kernel(x, token_table):
    x32 = x.astype(jnp.int32)
    pe = jnp.asarray(_PE)
    mesh = plsc.VectorSubcoreMesh(core_axis_name="c", subcore_axis_name="s")
    run = functools.partial(
        pl.kernel,
        mesh=mesh,
        out_type=jax.ShapeDtypeStruct((BATCH * SEQ, EMB_DIM), jnp.float32),
        scratch_types=[
            pltpu.VMEM((BATCH, _PPW), jnp.int32),
            pltpu.VMEM((_C, EMB_DIM), jnp.float32),
            pltpu.VMEM((_C, EMB_DIM), jnp.float32),
            pltpu.VMEM((_C, EMB_DIM), jnp.float32),
            pltpu.VMEM((_C, EMB_DIM), jnp.float32),
            pltpu.VMEM((_C, EMB_DIM), jnp.float32),
            pltpu.VMEM((_C, EMB_DIM), jnp.float32),
            pltpu.SemaphoreType.DMA,
            pltpu.SemaphoreType.DMA,
            pltpu.SemaphoreType.DMA,
            pltpu.SemaphoreType.DMA,
            pltpu.SemaphoreType.DMA,
            pltpu.SemaphoreType.DMA,
        ],
    )(_body)
    out = run(x32, token_table, pe)
    return out.reshape(BATCH, SEQ, EMB_DIM)
